# baseline (speedup 1.0000x reference)
import jax
import jax.numpy as jnp
from jax.experimental import pallas as pl
from jax.experimental.pallas import tpu as pltpu

B = 256
N = 100
NP = 104
IN_DIM = 16
INNER = 256
LATENT = 128
BB = 64


def _relu(v):
    return jnp.maximum(v, 0.0)


def _gat_block(h, w_ref, wa_ref, p_mat, residual):
    w = w_ref[...]
    z = jax.lax.dot_general(h, w, (((1,), (1,)), ((), ())),
                            preferred_element_type=jnp.float32)
    eff = jax.lax.dot_general(wa_ref[...], w, (((1,), (0,)), ((), ())),
                              preferred_element_type=jnp.float32)
    sc = jax.lax.dot_general(h, eff, (((1,), (1,)), ((), ())),
                             preferred_element_type=jnp.float32)
    es_col = sc[:, 0:1]
    ed_all = sc[:, 1:2]
    es_mat = es_col.reshape(BB, NP)
    colrow = jax.lax.broadcasted_iota(jnp.int32, (BB, NP), 1)
    es_mat = jnp.where(colrow < N, es_mat, -1e30)
    e = ed_all + jnp.dot(p_mat, es_mat,
                         preferred_element_type=jnp.float32)
    e = jnp.maximum(e, 0.01 * e)
    p = jnp.exp(e)
    s = jnp.sum(p, axis=1, keepdims=True)
    a = p * jax.lax.reciprocal(s)
    zl = z.astype(jnp.bfloat16)
    al = a.astype(jnp.bfloat16)
    outs = []
    for b in range(BB):
        zb = zl[b * NP:(b + 1) * NP, :]
        ab = al[b * NP:(b + 1) * NP, :]
        outs.append(jnp.dot(ab, zb, preferred_element_type=jnp.float32))
    out = jnp.concatenate(outs, axis=0)
    if residual:
        out = _relu(out + h)
    return out


def _encoder_kernel(x_ref, wr_ref, b_ref,
                    w1_ref, wa1_ref, w2_ref, wa2_ref,
                    w3_ref, wa3_ref, w4_ref, wa4_ref,
                    out_ref):
    r = jax.lax.broadcasted_iota(jnp.int32, (BB * NP, BB), 0) // NP
    c = jax.lax.broadcasted_iota(jnp.int32, (BB * NP, BB), 1)
    p_mat = (r == c).astype(jnp.float32)
    x2 = x_ref[...]
    zrow = jnp.zeros((NP - N, IN_DIM), dtype=jnp.float32)
    pieces = []
    for b in range(BB):
        pieces.append(x2[b * N:(b + 1) * N, :])
        pieces.append(zrow)
    xp = jnp.concatenate(pieces, axis=0)
    h = _relu(jax.lax.dot_general(xp, wr_ref[...],
                                  (((1,), (1,)), ((), ())),
                                  preferred_element_type=jnp.float32)
              + b_ref[...])
    h = _gat_block(h, w1_ref, wa1_ref, p_mat, True)
    h = _gat_block(h, w2_ref, wa2_ref, p_mat, True)
    h = _gat_block(h, w3_ref, wa3_ref, p_mat, True)
    out = _gat_block(h, w4_ref, wa4_ref, p_mat, False)
    for b in range(BB):
        out_ref[b, :, :] = out[b * NP:b * NP + N, :]


def kernel(x, W_remap, b_remap, Wfc1, Wa1, Wfc2, Wa2, Wfc3, Wa3, Wfc4, Wa4):
    x2 = x.reshape(B * N, IN_DIM)
    br = b_remap.reshape(1, INNER)

    a1 = Wa1.reshape(2, INNER)
    a2 = Wa2.reshape(2, INNER)
    a3 = Wa3.reshape(2, INNER)
    a4 = Wa4.reshape(2, LATENT)

    full = lambda shp: pl.BlockSpec(shp, lambda i: (0, 0))
    out = pl.pallas_call(
        _encoder_kernel,
        grid=(B // BB,),
        in_specs=[
            pl.BlockSpec((BB * N, IN_DIM), lambda i: (i, 0)),
            full((INNER, IN_DIM)), full((1, INNER)),
            full((INNER, INNER)), full((2, INNER)),
            full((INNER, INNER)), full((2, INNER)),
            full((INNER, INNER)), full((2, INNER)),
            full((LATENT, INNER)), full((2, LATENT)),
        ],
        out_specs=pl.BlockSpec((BB, N, LATENT), lambda i: (i, 0, 0)),
        out_shape=jax.ShapeDtypeStruct((B, N, LATENT), jnp.float32),
        compiler_params=pltpu.CompilerParams(
            dimension_semantics=("arbitrary",)),
    )(x2, W_remap, br, Wfc1, a1, Wfc2, a2, Wfc3, a3, Wfc4, a4)
    return out


# speedup vs baseline: 1.0140x; 1.0140x over previous
import jax
import jax.numpy as jnp
from jax.experimental import pallas as pl
from jax.experimental.pallas import tpu as pltpu

B = 256
N = 100
NP = 104
IN_DIM = 16
INNER = 256
LATENT = 128
BB = 64


def _relu(v):
    return jnp.maximum(v, 0.0)


def _gat_block(h, w_ref, wa_ref, p_mat, residual):
    w = w_ref[...]
    z = jax.lax.dot_general(h, w, (((1,), (1,)), ((), ())),
                            preferred_element_type=jnp.float32)
    eff = jax.lax.dot_general(wa_ref[...], w, (((1,), (0,)), ((), ())),
                              preferred_element_type=jnp.float32)
    sc = jax.lax.dot_general(h, eff, (((1,), (1,)), ((), ())),
                             preferred_element_type=jnp.float32)
    es_col = sc[:, 0:1]
    ed_all = sc[:, 1:2]
    es_mat = es_col.reshape(BB, NP)
    colrow = jax.lax.broadcasted_iota(jnp.int32, (BB, NP), 1)
    es_mat = jnp.where(colrow < N, es_mat, -1e30)
    e = ed_all + jnp.dot(p_mat, es_mat,
                         preferred_element_type=jnp.float32)
    e = jnp.maximum(e, 0.01 * e)
    p = jnp.exp(e)
    s = jnp.sum(p, axis=1, keepdims=True)
    a = p * jax.lax.reciprocal(s)
    if not residual:
        return a, z
    outs = []
    for b in range(BB):
        zb = z[b * NP:(b + 1) * NP, :]
        ab = a[b * NP:(b + 1) * NP, :]
        outs.append(jnp.dot(ab, zb, preferred_element_type=jnp.float32))
    return _relu(jnp.concatenate(outs, axis=0) + h)


def _encoder_kernel(x_ref, wr_ref, b_ref,
                    w1_ref, wa1_ref, w2_ref, wa2_ref,
                    w3_ref, wa3_ref, w4_ref, wa4_ref,
                    out_ref):
    r = jax.lax.broadcasted_iota(jnp.int32, (BB * NP, BB), 0) // NP
    c = jax.lax.broadcasted_iota(jnp.int32, (BB * NP, BB), 1)
    p_mat = (r == c).astype(jnp.float32)
    x2 = x_ref[...]
    zrow = jnp.zeros((NP - N, IN_DIM), dtype=jnp.float32)
    pieces = []
    for b in range(BB):
        pieces.append(x2[b * N:(b + 1) * N, :])
        pieces.append(zrow)
    xp = jnp.concatenate(pieces, axis=0)
    h = _relu(jax.lax.dot_general(xp, wr_ref[...],
                                  (((1,), (1,)), ((), ())),
                                  preferred_element_type=jnp.float32)
              + b_ref[...])
    h = _gat_block(h, w1_ref, wa1_ref, p_mat, True)
    h = _gat_block(h, w2_ref, wa2_ref, p_mat, True)
    h = _gat_block(h, w3_ref, wa3_ref, p_mat, True)
    a4, z4 = _gat_block(h, w4_ref, wa4_ref, p_mat, False)
    for b in range(BB):
        ab = a4[b * NP:b * NP + N, :]
        zb = z4[b * NP:(b + 1) * NP, :]
        out_ref[b, :, :] = jnp.dot(ab, zb, preferred_element_type=jnp.float32)


def kernel(x, W_remap, b_remap, Wfc1, Wa1, Wfc2, Wa2, Wfc3, Wa3, Wfc4, Wa4):
    x2 = x.reshape(B * N, IN_DIM)
    br = b_remap.reshape(1, INNER)

    a1 = Wa1.reshape(2, INNER)
    a2 = Wa2.reshape(2, INNER)
    a3 = Wa3.reshape(2, INNER)
    a4 = Wa4.reshape(2, LATENT)

    full = lambda shp: pl.BlockSpec(shp, lambda i: (0, 0))
    out = pl.pallas_call(
        _encoder_kernel,
        grid=(B // BB,),
        in_specs=[
            pl.BlockSpec((BB * N, IN_DIM), lambda i: (i, 0)),
            full((INNER, IN_DIM)), full((1, INNER)),
            full((INNER, INNER)), full((2, INNER)),
            full((INNER, INNER)), full((2, INNER)),
            full((INNER, INNER)), full((2, INNER)),
            full((LATENT, INNER)), full((2, LATENT)),
        ],
        out_specs=pl.BlockSpec((BB, N, LATENT), lambda i: (i, 0, 0)),
        out_shape=jax.ShapeDtypeStruct((B, N, LATENT), jnp.float32),
        compiler_params=pltpu.CompilerParams(
            dimension_semantics=("arbitrary",)),
    )(x2, W_remap, br, Wfc1, a1, Wfc2, a2, Wfc3, a3, Wfc4, a4)
    return out
